# baseline (speedup 1.0000x reference)
import functools
import jax
import jax.numpy as jnp
from jax import lax
from jax.experimental import pallas as pl
from jax.experimental.pallas import tpu as pltpu
from jax.experimental.pallas import tpu_sc as plsc

D_MODEL_K = 64
SCALE = 8.0
NUM_CORES = 2
NUM_SUBCORES = 16
NUM_WORKERS = NUM_CORES * NUM_SUBCORES
CHUNK = 512


def _make_sc_gather(B):
    assert B % (8 * NUM_WORKERS) == 0
    b_per_w = B // NUM_WORKERS
    assert b_per_w % CHUNK == 0
    n_chunks = b_per_w // CHUNK
    mesh = plsc.VectorSubcoreMesh(core_axis_name="c", subcore_axis_name="s")

    @functools.partial(
        pl.kernel,
        mesh=mesh,
        out_type=jax.ShapeDtypeStruct((B, D_MODEL_K), jnp.float32),
        scratch_types=[
            pltpu.VMEM((b_per_w,), jnp.int32),
            pltpu.VMEM((CHUNK, D_MODEL_K), jnp.float32),
            pltpu.SemaphoreType.DMA,
        ],
        compiler_params=pltpu.CompilerParams(use_tc_tiling_on_sc=False),
    )
    def emb_kernel(idx_hbm, table_hbm, out_hbm, idx_v, rows_v, sem):
        wid = lax.axis_index("s") * NUM_CORES + lax.axis_index("c")
        base = wid * b_per_w
        pltpu.sync_copy(idx_hbm.at[pl.ds(base, b_per_w)], idx_v)

        def chunk_body(c, carry):
            start = c * CHUNK
            pltpu.async_copy(
                table_hbm.at[idx_v.at[pl.ds(start, CHUNK)]], rows_v, sem
            ).wait()

            def row_body(i, carry2):
                for j in range(D_MODEL_K // 16):
                    sl = pl.ds(j * 16, 16)
                    rows_v[i, sl] = rows_v[i, sl] * SCALE
                return carry2

            lax.fori_loop(0, CHUNK, row_body, 0, unroll=2)
            pltpu.sync_copy(rows_v, out_hbm.at[pl.ds(base + start, CHUNK)])
            return carry

        lax.fori_loop(0, n_chunks, chunk_body, 0)

    return emb_kernel


def kernel(input_tensor, table):
    rows, cols = input_tensor.shape
    B = rows * cols
    idx = input_tensor.reshape(B).astype(jnp.int32)
    out = _make_sc_gather(B)(idx, table)
    return out.reshape(rows, cols, D_MODEL_K)


# speedup vs baseline: 1.0730x; 1.0730x over previous
import functools
import jax
import jax.numpy as jnp
from jax import lax
from jax.experimental import pallas as pl
from jax.experimental.pallas import tpu as pltpu
from jax.experimental.pallas import tpu_sc as plsc

D_MODEL_K = 64
SCALE = 8.0
NUM_CORES = 2
NUM_SUBCORES = 16
NUM_WORKERS = NUM_CORES * NUM_SUBCORES
CHUNK = 512


def _make_sc_gather(B):
    assert B % (8 * NUM_WORKERS) == 0
    b_per_w = B // NUM_WORKERS
    assert b_per_w % CHUNK == 0
    n_chunks = b_per_w // CHUNK
    assert n_chunks >= 4 and n_chunks % 2 == 0
    mesh = plsc.VectorSubcoreMesh(core_axis_name="c", subcore_axis_name="s")

    @functools.partial(
        pl.kernel,
        mesh=mesh,
        out_type=jax.ShapeDtypeStruct((B, D_MODEL_K), jnp.float32),
        scratch_types=[
            pltpu.VMEM((b_per_w,), jnp.int32),
            pltpu.VMEM((CHUNK, D_MODEL_K), jnp.float32),
            pltpu.VMEM((CHUNK, D_MODEL_K), jnp.float32),
            pltpu.SemaphoreType.DMA,
            pltpu.SemaphoreType.DMA,
            pltpu.SemaphoreType.DMA,
            pltpu.SemaphoreType.DMA,
        ],
        compiler_params=pltpu.CompilerParams(use_tc_tiling_on_sc=False),
    )
    def emb_kernel(idx_hbm, table_hbm, out_hbm, idx_v, rows0, rows1,
                   g0, g1, s0, s1):
        wid = lax.axis_index("s") * NUM_CORES + lax.axis_index("c")
        base = wid * b_per_w
        pltpu.sync_copy(idx_hbm.at[pl.ds(base, b_per_w)], idx_v)

        bufs = (rows0, rows1)
        gsems = (g0, g1)
        ssems = (s0, s1)

        def start_gather(c, buf, sem):
            pltpu.async_copy(
                table_hbm.at[idx_v.at[pl.ds(c * CHUNK, CHUNK)]], buf, sem
            )

        def scale(buf):
            def row_body(i, carry2):
                for j in range(D_MODEL_K // 16):
                    sl = pl.ds(j * 16, 16)
                    buf[i, sl] = buf[i, sl] * SCALE
                return carry2

            lax.fori_loop(0, CHUNK, row_body, 0, unroll=4)

        def start_scatter(c, buf, sem):
            pltpu.async_copy(buf, out_hbm.at[pl.ds(base + c * CHUNK, CHUNK)], sem)

        def wait_gather(c, buf, sem):
            pltpu.make_async_copy(
                table_hbm.at[idx_v.at[pl.ds(c * CHUNK, CHUNK)]], buf, sem
            ).wait()

        def wait_scatter(c, buf, sem):
            pltpu.make_async_copy(
                buf, out_hbm.at[pl.ds(base + c * CHUNK, CHUNK)], sem
            ).wait()

        start_gather(0, bufs[0], gsems[0])
        start_gather(1, bufs[1], gsems[1])
        wait_gather(0, bufs[0], gsems[0])
        scale(bufs[0])
        start_scatter(0, bufs[0], ssems[0])

        def outer(p, carry):
            g = 1 + 2 * p
            for b in range(2):
                c = g + b
                pb = (1 + b) % 2
                ob = (pb + 1) % 2
                wait_scatter(c - 1, bufs[ob], ssems[ob])
                start_gather(c + 1, bufs[ob], gsems[ob])
                wait_gather(c, bufs[pb], gsems[pb])
                scale(bufs[pb])
                start_scatter(c, bufs[pb], ssems[pb])
            return carry

        lax.fori_loop(0, (n_chunks - 2) // 2, outer, 0)

        lb = (n_chunks - 1) % 2
        wait_gather(n_chunks - 1, bufs[lb], gsems[lb])
        scale(bufs[lb])
        start_scatter(n_chunks - 1, bufs[lb], ssems[lb])
        wait_scatter(n_chunks - 2, bufs[1 - lb], ssems[1 - lb])
        wait_scatter(n_chunks - 1, bufs[lb], ssems[lb])

    return emb_kernel


def kernel(input_tensor, table):
    rows, cols = input_tensor.shape
    B = rows * cols
    idx = input_tensor.reshape(B).astype(jnp.int32)
    out = _make_sc_gather(B)(idx, table)
    return out.reshape(rows, cols, D_MODEL_K)
